# baseline (speedup 1.0000x reference)
import functools

import jax
import jax.numpy as jnp
from jax import lax
from jax.experimental import pallas as pl
from jax.experimental.pallas import tpu as pltpu
from jax.experimental.pallas import tpu_sc as plsc

BATCH = 16384
DIM = 32
NC = 2
NS = 16
LANES = 16
NW = NC * NS
BPW = BATCH // NW
CH = 128
NCHUNK = BPW // CH
LINE = 128

TBLK = 2048
GRP = TBLK // 4

_mesh = plsc.VectorSubcoreMesh(core_axis_name="c", subcore_axis_name="s")


def _transpose_body(tab_t_ref, out_ref):
    x = tab_t_ref[...]
    eye = jnp.eye(DIM, dtype=jnp.float32)
    y = lax.dot_general(x, eye, (((0,), (0,)), ((), ())),
                        precision=lax.Precision.HIGHEST,
                        preferred_element_type=jnp.float32)
    out_ref[...] = jnp.concatenate(
        [y[c * GRP:(c + 1) * GRP, :] for c in range(4)], axis=1)


def _to_lines(tab_t):
    n = tab_t.shape[1]
    grid = pl.cdiv(n, TBLK)
    return pl.pallas_call(
        _transpose_body,
        grid=(grid,),
        in_specs=[pl.BlockSpec((DIM, TBLK), lambda g: (0, g))],
        out_specs=pl.BlockSpec((GRP, LINE), lambda g: (g, 0)),
        out_shape=jax.ShapeDtypeStruct((grid * GRP, LINE), jnp.float32),
    )(tab_t)


def _line_of(r):
    return lax.shift_left(lax.shift_right_logical(r, 11), 9) | (r & 511)


def _colbase_of(r):
    return lax.shift_left(lax.shift_right_logical(r, 9) & 3, 5)


@functools.partial(
    pl.kernel,
    out_type=jax.ShapeDtypeStruct((BATCH,), jnp.float32),
    mesh=_mesh,
    compiler_params=pltpu.CompilerParams(needs_layout_passes=False),
    scratch_types=[
        pltpu.VMEM((NCHUNK, CH), jnp.int32),
        pltpu.VMEM((NCHUNK, CH), jnp.int32),
        pltpu.VMEM((NCHUNK, CH), jnp.int32),
        pltpu.VMEM((NCHUNK, CH), jnp.int32),
        pltpu.VMEM((2, CH, LINE), jnp.float32),
        pltpu.VMEM((2, CH, LINE), jnp.float32),
        pltpu.VMEM((BPW,), jnp.float32),
        pltpu.SemaphoreType.DMA,
        pltpu.SemaphoreType.DMA,
        pltpu.SemaphoreType.DMA,
        pltpu.SemaphoreType.DMA,
        pltpu.SemaphoreType.DMA,
    ],
)
def _mf_sc(user_hbm, item_hbm, utab_hbm, itab_hbm, out_hbm,
           uidx_v, iidx_v, ugidx_v, igidx_v, ubuf, ibuf, res_v,
           sem_idx, sem_u0, sem_u1, sem_i0, sem_i1):
    wid = lax.axis_index("s") * NC + lax.axis_index("c")
    base = wid * BPW

    idx_copies = []
    for j in range(NCHUNK):
        off = base + j * CH
        idx_copies.append(pltpu.async_copy(
            user_hbm.at[pl.ds(off, CH)], uidx_v.at[j], sem_idx))
        idx_copies.append(pltpu.async_copy(
            item_hbm.at[pl.ds(off, CH)], iidx_v.at[j], sem_idx))
    for c in idx_copies:
        c.wait()

    lane = lax.iota(jnp.int32, LANES)
    for j in range(NCHUNK):
        jf = jnp.full((LANES,), j, jnp.int32)
        for q in range(CH // LANES):
            kq = lane + q * LANES
            ru = plsc.load_gather(uidx_v, [jf, kq])
            ri = plsc.load_gather(iidx_v, [jf, kq])
            plsc.store_scatter(ugidx_v, [jf, kq], _line_of(ru))
            plsc.store_scatter(igidx_v, [jf, kq], _line_of(ri))

    sem_u = (sem_u0, sem_u1)
    sem_i = (sem_i0, sem_i1)

    def fire(j):
        s = j & 1
        return (pltpu.async_copy(utab_hbm.at[ugidx_v.at[j]], ubuf.at[s],
                                 sem_u[s]),
                pltpu.async_copy(itab_hbm.at[igidx_v.at[j]], ibuf.at[s],
                                 sem_i[s]))

    pending = fire(0)
    for j in range(NCHUNK):
        cu_, ci_ = pending
        if j + 1 < NCHUNK:
            nxt = fire(j + 1)
        cu_.wait()
        ci_.wait()
        if j + 1 < NCHUNK:
            pending = nxt

        s = j & 1
        ub2 = ubuf.at[s]
        ib2 = ibuf.at[s]
        jf = jnp.full((LANES,), j, jnp.int32)

        def qbody(q, carry, ub2=ub2, ib2=ib2, jf=jf):
            kq = lane + q * LANES
            ru = plsc.load_gather(uidx_v, [jf, kq])
            ri = plsc.load_gather(iidx_v, [jf, kq])
            ucol = _colbase_of(ru)
            icol = _colbase_of(ri)
            acc = jnp.zeros((LANES,), jnp.float32)
            for d in range(DIM):
                u = plsc.load_gather(ub2, [kq, ucol + d])
                v = plsc.load_gather(ib2, [kq, icol + d])
                acc = acc + u * v
            plsc.store_scatter(res_v, [jf * CH + kq], acc)
            return carry

        lax.fori_loop(0, CH // LANES, qbody, 0)

    pltpu.sync_copy(res_v, out_hbm.at[pl.ds(base, BPW)])


def kernel(user, item, user_emb_table, item_emb_table):
    utab = _to_lines(user_emb_table.T)
    itab = _to_lines(item_emb_table.T)
    return _mf_sc(user.astype(jnp.int32), item.astype(jnp.int32),
                  utab, itab)


# speedup vs baseline: 1.3501x; 1.3501x over previous
import functools

import jax
import jax.numpy as jnp
from jax import lax
from jax.experimental import pallas as pl
from jax.experimental.pallas import tpu as pltpu
from jax.experimental.pallas import tpu_sc as plsc

BATCH = 16384
DIM = 32
NC = 2
NS = 16
LANES = 16
NW = NC * NS
BPW = BATCH // NW
CH = 128
NCHUNK = BPW // CH
LINE = 128

TBLK = 2048
GRP = TBLK // 4

_mesh = plsc.VectorSubcoreMesh(core_axis_name="c", subcore_axis_name="s")


def _transpose_body(tab_t_ref, out_ref):
    x = tab_t_ref[...]
    eye = jnp.eye(DIM, dtype=jnp.float32)
    y = lax.dot_general(x, eye, (((0,), (0,)), ((), ())),
                        precision=lax.Precision.DEFAULT,
                        preferred_element_type=jnp.float32)
    out_ref[...] = jnp.concatenate(
        [y[c * GRP:(c + 1) * GRP, :] for c in range(4)], axis=1)


def _to_lines(tab_t):
    n = tab_t.shape[1]
    grid = pl.cdiv(n, TBLK)
    return pl.pallas_call(
        _transpose_body,
        grid=(grid,),
        in_specs=[pl.BlockSpec((DIM, TBLK), lambda g: (0, g))],
        out_specs=pl.BlockSpec((GRP, LINE), lambda g: (g, 0)),
        out_shape=jax.ShapeDtypeStruct((grid * GRP, LINE), jnp.float32),
    )(tab_t)


def _line_of(r):
    return lax.shift_left(lax.shift_right_logical(r, 11), 9) | (r & 511)


def _colbase_of(r):
    return lax.shift_left(lax.shift_right_logical(r, 9) & 3, 5)


@functools.partial(
    pl.kernel,
    out_type=jax.ShapeDtypeStruct((BATCH,), jnp.float32),
    mesh=_mesh,
    compiler_params=pltpu.CompilerParams(needs_layout_passes=False),
    scratch_types=[
        pltpu.VMEM((NCHUNK, CH), jnp.int32),
        pltpu.VMEM((NCHUNK, CH), jnp.int32),
        pltpu.VMEM((NCHUNK, CH), jnp.int32),
        pltpu.VMEM((NCHUNK, CH), jnp.int32),
        pltpu.VMEM((2, CH, LINE), jnp.float32),
        pltpu.VMEM((2, CH, LINE), jnp.float32),
        pltpu.VMEM((BPW,), jnp.float32),
        pltpu.SemaphoreType.DMA,
        pltpu.SemaphoreType.DMA,
        pltpu.SemaphoreType.DMA,
        pltpu.SemaphoreType.DMA,
        pltpu.SemaphoreType.DMA,
    ],
)
def _mf_sc(user_hbm, item_hbm, utab_hbm, itab_hbm, out_hbm,
           uidx_v, iidx_v, ugidx_v, igidx_v, ubuf, ibuf, res_v,
           sem_idx, sem_u0, sem_u1, sem_i0, sem_i1):
    wid = lax.axis_index("s") * NC + lax.axis_index("c")
    base = wid * BPW

    idx_copies = []
    for j in range(NCHUNK):
        off = base + j * CH
        idx_copies.append(pltpu.async_copy(
            user_hbm.at[pl.ds(off, CH)], uidx_v.at[j], sem_idx))
        idx_copies.append(pltpu.async_copy(
            item_hbm.at[pl.ds(off, CH)], iidx_v.at[j], sem_idx))
    for c in idx_copies:
        c.wait()

    lane = lax.iota(jnp.int32, LANES)
    for j in range(NCHUNK):
        jf = jnp.full((LANES,), j, jnp.int32)
        for q in range(CH // LANES):
            kq = lane + q * LANES
            ru = plsc.load_gather(uidx_v, [jf, kq])
            ri = plsc.load_gather(iidx_v, [jf, kq])
            plsc.store_scatter(ugidx_v, [jf, kq], _line_of(ru))
            plsc.store_scatter(igidx_v, [jf, kq], _line_of(ri))

    sem_u = (sem_u0, sem_u1)
    sem_i = (sem_i0, sem_i1)

    def fire(j):
        s = j & 1
        return (pltpu.async_copy(utab_hbm.at[ugidx_v.at[j]], ubuf.at[s],
                                 sem_u[s]),
                pltpu.async_copy(itab_hbm.at[igidx_v.at[j]], ibuf.at[s],
                                 sem_i[s]))

    pending = fire(0)
    for j in range(NCHUNK):
        cu_, ci_ = pending
        if j + 1 < NCHUNK:
            nxt = fire(j + 1)
        cu_.wait()
        ci_.wait()
        if j + 1 < NCHUNK:
            pending = nxt

        s = j & 1
        ub2 = ubuf.at[s]
        ib2 = ibuf.at[s]
        jf = jnp.full((LANES,), j, jnp.int32)

        def qbody(q, carry, ub2=ub2, ib2=ib2, jf=jf):
            kq = lane + q * LANES
            ru = plsc.load_gather(uidx_v, [jf, kq])
            ri = plsc.load_gather(iidx_v, [jf, kq])
            ucol = _colbase_of(ru)
            icol = _colbase_of(ri)
            acc = jnp.zeros((LANES,), jnp.float32)
            for d in range(DIM):
                u = plsc.load_gather(ub2, [kq, ucol + d])
                v = plsc.load_gather(ib2, [kq, icol + d])
                acc = acc + u * v
            plsc.store_scatter(res_v, [jf * CH + kq], acc)
            return carry

        lax.fori_loop(0, CH // LANES, qbody, 0)

    pltpu.sync_copy(res_v, out_hbm.at[pl.ds(base, BPW)])


def kernel(user, item, user_emb_table, item_emb_table):
    utab = _to_lines(user_emb_table.T)
    itab = _to_lines(item_emb_table.T)
    return _mf_sc(user.astype(jnp.int32), item.astype(jnp.int32),
                  utab, itab)


# speedup vs baseline: 2.2075x; 1.6350x over previous
import functools

import jax
import jax.numpy as jnp
from jax import lax
from jax.experimental import pallas as pl
from jax.experimental.pallas import tpu as pltpu
from jax.experimental.pallas import tpu_sc as plsc

BATCH = 16384
DIM = 32
NC = 2
NS = 16
LANES = 16
NW = NC * NS
BPW = BATCH // NW
CH = 128
NCHUNK = BPW // CH
LINE = 128

TBLK = 16384
GRP = TBLK // 4
TSH = 14
GSH = 12

_mesh = plsc.VectorSubcoreMesh(core_axis_name="c", subcore_axis_name="s")


def _transpose_body(tab_t_ref, out_ref):
    y = tab_t_ref[...].T
    out_ref[...] = jnp.concatenate(
        [y[c * GRP:(c + 1) * GRP, :] for c in range(4)], axis=1)


def _to_lines(tab_t):
    n = tab_t.shape[1]
    grid = pl.cdiv(n, TBLK)
    return pl.pallas_call(
        _transpose_body,
        grid=(grid,),
        in_specs=[pl.BlockSpec((DIM, TBLK), lambda g: (0, g))],
        out_specs=pl.BlockSpec((GRP, LINE), lambda g: (g, 0)),
        out_shape=jax.ShapeDtypeStruct((grid * GRP, LINE), jnp.float32),
    )(tab_t)


def _line_of(r):
    return (lax.shift_left(lax.shift_right_logical(r, TSH), GSH)
            | (r & (GRP - 1)))


def _colbase_of(r):
    return lax.shift_left(lax.shift_right_logical(r, GSH) & 3, 5)


@functools.partial(
    pl.kernel,
    out_type=jax.ShapeDtypeStruct((BATCH,), jnp.float32),
    mesh=_mesh,
    compiler_params=pltpu.CompilerParams(needs_layout_passes=False),
    scratch_types=[
        pltpu.VMEM((NCHUNK, CH), jnp.int32),
        pltpu.VMEM((NCHUNK, CH), jnp.int32),
        pltpu.VMEM((NCHUNK, CH), jnp.int32),
        pltpu.VMEM((NCHUNK, CH), jnp.int32),
        pltpu.VMEM((2, CH, LINE), jnp.float32),
        pltpu.VMEM((2, CH, LINE), jnp.float32),
        pltpu.VMEM((BPW,), jnp.float32),
        pltpu.SemaphoreType.DMA,
        pltpu.SemaphoreType.DMA,
        pltpu.SemaphoreType.DMA,
        pltpu.SemaphoreType.DMA,
        pltpu.SemaphoreType.DMA,
    ],
)
def _mf_sc(user_hbm, item_hbm, utab_hbm, itab_hbm, out_hbm,
           uidx_v, iidx_v, ugidx_v, igidx_v, ubuf, ibuf, res_v,
           sem_idx, sem_u0, sem_u1, sem_i0, sem_i1):
    wid = lax.axis_index("s") * NC + lax.axis_index("c")
    base = wid * BPW

    idx_copies = []
    for j in range(NCHUNK):
        off = base + j * CH
        idx_copies.append(pltpu.async_copy(
            user_hbm.at[pl.ds(off, CH)], uidx_v.at[j], sem_idx))
        idx_copies.append(pltpu.async_copy(
            item_hbm.at[pl.ds(off, CH)], iidx_v.at[j], sem_idx))
    for c in idx_copies:
        c.wait()

    lane = lax.iota(jnp.int32, LANES)
    for j in range(NCHUNK):
        jf = jnp.full((LANES,), j, jnp.int32)
        for q in range(CH // LANES):
            kq = lane + q * LANES
            ru = plsc.load_gather(uidx_v, [jf, kq])
            ri = plsc.load_gather(iidx_v, [jf, kq])
            plsc.store_scatter(ugidx_v, [jf, kq], _line_of(ru))
            plsc.store_scatter(igidx_v, [jf, kq], _line_of(ri))

    sem_u = (sem_u0, sem_u1)
    sem_i = (sem_i0, sem_i1)

    def fire(j):
        s = j & 1
        return (pltpu.async_copy(utab_hbm.at[ugidx_v.at[j]], ubuf.at[s],
                                 sem_u[s]),
                pltpu.async_copy(itab_hbm.at[igidx_v.at[j]], ibuf.at[s],
                                 sem_i[s]))

    pending = fire(0)
    for j in range(NCHUNK):
        cu_, ci_ = pending
        if j + 1 < NCHUNK:
            nxt = fire(j + 1)
        cu_.wait()
        ci_.wait()
        if j + 1 < NCHUNK:
            pending = nxt

        s = j & 1
        ub2 = ubuf.at[s]
        ib2 = ibuf.at[s]
        jf = jnp.full((LANES,), j, jnp.int32)

        def qbody(q, carry, ub2=ub2, ib2=ib2, jf=jf):
            kq = lane + q * LANES
            ru = plsc.load_gather(uidx_v, [jf, kq])
            ri = plsc.load_gather(iidx_v, [jf, kq])
            ucol = _colbase_of(ru)
            icol = _colbase_of(ri)
            acc = jnp.zeros((LANES,), jnp.float32)
            for d in range(DIM):
                u = plsc.load_gather(ub2, [kq, ucol + d])
                v = plsc.load_gather(ib2, [kq, icol + d])
                acc = acc + u * v
            plsc.store_scatter(res_v, [jf * CH + kq], acc)
            return carry

        lax.fori_loop(0, CH // LANES, qbody, 0)

    pltpu.sync_copy(res_v, out_hbm.at[pl.ds(base, BPW)])


def kernel(user, item, user_emb_table, item_emb_table):
    utab = _to_lines(user_emb_table.T)
    itab = _to_lines(item_emb_table.T)
    return _mf_sc(user.astype(jnp.int32), item.astype(jnp.int32),
                  utab, itab)


# speedup vs baseline: 2.2251x; 1.0080x over previous
import functools

import jax
import jax.numpy as jnp
from jax import lax
from jax.experimental import pallas as pl
from jax.experimental.pallas import tpu as pltpu
from jax.experimental.pallas import tpu_sc as plsc

BATCH = 16384
DIM = 32
NC = 2
NS = 16
LANES = 16
NW = NC * NS
BPW = BATCH // NW
CH = 128
NCHUNK = BPW // CH
LINE = 128

TBLK = 32768
GRP = TBLK // 4
TSH = 15
GSH = 13

_mesh = plsc.VectorSubcoreMesh(core_axis_name="c", subcore_axis_name="s")


def _transpose_body(tab_t_ref, out_ref):
    y = tab_t_ref[...].T
    out_ref[...] = jnp.concatenate(
        [y[c * GRP:(c + 1) * GRP, :] for c in range(4)], axis=1)


def _to_lines(tab_t):
    n = tab_t.shape[1]
    grid = pl.cdiv(n, TBLK)
    return pl.pallas_call(
        _transpose_body,
        grid=(grid,),
        in_specs=[pl.BlockSpec((DIM, TBLK), lambda g: (0, g))],
        out_specs=pl.BlockSpec((GRP, LINE), lambda g: (g, 0)),
        out_shape=jax.ShapeDtypeStruct((grid * GRP, LINE), jnp.float32),
    )(tab_t)


def _line_of(r):
    return (lax.shift_left(lax.shift_right_logical(r, TSH), GSH)
            | (r & (GRP - 1)))


def _colbase_of(r):
    return lax.shift_left(lax.shift_right_logical(r, GSH) & 3, 5)


@functools.partial(
    pl.kernel,
    out_type=jax.ShapeDtypeStruct((BATCH,), jnp.float32),
    mesh=_mesh,
    compiler_params=pltpu.CompilerParams(needs_layout_passes=False),
    scratch_types=[
        pltpu.VMEM((NCHUNK, CH), jnp.int32),
        pltpu.VMEM((NCHUNK, CH), jnp.int32),
        pltpu.VMEM((NCHUNK, CH), jnp.int32),
        pltpu.VMEM((NCHUNK, CH), jnp.int32),
        pltpu.VMEM((2, CH, LINE), jnp.float32),
        pltpu.VMEM((2, CH, LINE), jnp.float32),
        pltpu.VMEM((BPW,), jnp.float32),
        pltpu.SemaphoreType.DMA,
        pltpu.SemaphoreType.DMA,
        pltpu.SemaphoreType.DMA,
        pltpu.SemaphoreType.DMA,
        pltpu.SemaphoreType.DMA,
    ],
)
def _mf_sc(user_hbm, item_hbm, utab_hbm, itab_hbm, out_hbm,
           uidx_v, iidx_v, ugidx_v, igidx_v, ubuf, ibuf, res_v,
           sem_idx, sem_u0, sem_u1, sem_i0, sem_i1):
    wid = lax.axis_index("s") * NC + lax.axis_index("c")
    base = wid * BPW

    idx_copies = []
    for j in range(NCHUNK):
        off = base + j * CH
        idx_copies.append(pltpu.async_copy(
            user_hbm.at[pl.ds(off, CH)], uidx_v.at[j], sem_idx))
        idx_copies.append(pltpu.async_copy(
            item_hbm.at[pl.ds(off, CH)], iidx_v.at[j], sem_idx))
    for c in idx_copies:
        c.wait()

    lane = lax.iota(jnp.int32, LANES)
    for j in range(NCHUNK):
        jf = jnp.full((LANES,), j, jnp.int32)
        for q in range(CH // LANES):
            kq = lane + q * LANES
            ru = plsc.load_gather(uidx_v, [jf, kq])
            ri = plsc.load_gather(iidx_v, [jf, kq])
            plsc.store_scatter(ugidx_v, [jf, kq], _line_of(ru))
            plsc.store_scatter(igidx_v, [jf, kq], _line_of(ri))

    sem_u = (sem_u0, sem_u1)
    sem_i = (sem_i0, sem_i1)

    def fire(j):
        s = j & 1
        return (pltpu.async_copy(utab_hbm.at[ugidx_v.at[j]], ubuf.at[s],
                                 sem_u[s]),
                pltpu.async_copy(itab_hbm.at[igidx_v.at[j]], ibuf.at[s],
                                 sem_i[s]))

    pending = fire(0)
    for j in range(NCHUNK):
        cu_, ci_ = pending
        if j + 1 < NCHUNK:
            nxt = fire(j + 1)
        cu_.wait()
        ci_.wait()
        if j + 1 < NCHUNK:
            pending = nxt

        s = j & 1
        ub2 = ubuf.at[s]
        ib2 = ibuf.at[s]
        jf = jnp.full((LANES,), j, jnp.int32)

        def qbody(q, carry, ub2=ub2, ib2=ib2, jf=jf):
            kq = lane + q * LANES
            ru = plsc.load_gather(uidx_v, [jf, kq])
            ri = plsc.load_gather(iidx_v, [jf, kq])
            ucol = _colbase_of(ru)
            icol = _colbase_of(ri)
            acc = jnp.zeros((LANES,), jnp.float32)
            for d in range(DIM):
                u = plsc.load_gather(ub2, [kq, ucol + d])
                v = plsc.load_gather(ib2, [kq, icol + d])
                acc = acc + u * v
            plsc.store_scatter(res_v, [jf * CH + kq], acc)
            return carry

        lax.fori_loop(0, CH // LANES, qbody, 0)

    pltpu.sync_copy(res_v, out_hbm.at[pl.ds(base, BPW)])


def kernel(user, item, user_emb_table, item_emb_table):
    utab = _to_lines(user_emb_table.T)
    itab = _to_lines(item_emb_table.T)
    return _mf_sc(user.astype(jnp.int32), item.astype(jnp.int32),
                  utab, itab)
